# baseline (speedup 1.0000x reference)
import functools

import jax
import jax.numpy as jnp
from jax import lax
from jax.experimental import pallas as pl
from jax.experimental.pallas import tpu as pltpu
from jax.experimental.pallas import tpu_sc as plsc

N = 10000
D = 128
E = 320000

NC = 2
NS = 16
CH = 128
NCH = 158
EP = NS * NCH * CH
NP = 10112
ZR = NP // NS

@functools.cache
def _get_mesh():
    return plsc.VectorSubcoreMesh(core_axis_name="c", subcore_axis_name="s",
                                  num_cores=NC)


def _run_job(table, ei, idx_s, idx_d, bb0, bb1, fb0, fb1, acc,
             g0, g1, s0, s1):
    w = lax.axis_index("s")
    pltpu.sync_copy(ei.at[0, w], idx_s)
    pltpu.sync_copy(ei.at[1, w], idx_d)

    hi_mask = jnp.int32(-65536)

    def g_start(j, buf, sem):
        pltpu.async_copy(table.at[idx_s.at[j]], buf, sem)

    def g_wait(j, buf, sem):
        pltpu.make_async_copy(table.at[idx_s.at[j]], buf, sem).wait()

    def s_start(j, buf, sem):
        pltpu.async_copy(buf, acc.at[idx_d.at[j]], sem, add=True)

    def s_wait(j, buf, sem):
        pltpu.make_async_copy(buf, acc.at[idx_d.at[j]], sem).wait()

    def widen(bb, fb):
        @pl.loop(0, CH, step=8)
        def _(r):
            for rr in range(8):
                for k in range(2):
                    v = bb[r + rr, pl.ds(16 * k, 16)]
                    lo = plsc.bitcast(lax.shift_left(v, 16), jnp.float32)
                    hi = plsc.bitcast(v & hi_mask, jnp.float32)
                    fb[r + rr, pl.ds(16 * k, 16)] = lo
                    fb[r + rr, pl.ds(16 * k + 32, 16)] = hi

    g_start(0, bb0, g0)
    g_start(1, bb1, g1)

    @pl.loop(0, NCH - 2, step=2)
    def _(j):
        g_wait(j, bb0, g0)
        widen(bb0, fb0)
        s_start(j, fb0, s0)
        g_wait(j + 1, bb1, g1)
        widen(bb1, fb1)
        s_start(j + 1, fb1, s1)
        s_wait(j, fb0, s0)
        g_start(j + 2, bb0, g0)
        s_wait(j + 1, fb1, s1)
        g_start(j + 3, bb1, g1)

    j = NCH - 2
    g_wait(j, bb0, g0)
    widen(bb0, fb0)
    s_start(j, fb0, s0)
    g_wait(j + 1, bb1, g1)
    widen(bb1, fb1)
    s_start(j + 1, fb1, s1)
    s_wait(j, fb0, s0)
    s_wait(j + 1, fb1, s1)


def _cnt_kernel(eiu, eii, z16, o16):
    f32 = jnp.float32
    outs = [jax.ShapeDtypeStruct((NP, 16), f32)] * 2
    scratch = [
        pltpu.VMEM((NCH, CH), jnp.int32),
        pltpu.VMEM((CH, 16), f32),
        pltpu.VMEM_SHARED((NP, 16), f32),
    ] + [pltpu.SemaphoreType.DMA] * 2

    @functools.partial(pl.kernel, out_type=outs, mesh=_get_mesh(),
                       scratch_types=scratch,
                       compiler_params=pltpu.CompilerParams(
                           use_tc_tiling_on_sc=False))
    def cntk(eiu, eii, z16, o16, cnt_ui, cnt_iu,
             idx_d, ones_v, cnt_acc, c0, c1):
        c = lax.axis_index("c")
        w = lax.axis_index("s")
        zsl = pl.ds(w * ZR, ZR)
        pltpu.sync_copy(z16, cnt_acc.at[zsl])
        pltpu.sync_copy(o16, ones_v)

        @pl.when(c == 0)
        def _():
            pltpu.sync_copy(eiu.at[1, w], idx_d)

        @pl.when(c == 1)
        def _():
            pltpu.sync_copy(eii.at[1, w], idx_d)

        plsc.subcore_barrier()

        def c_start(j, sem):
            pltpu.async_copy(ones_v, cnt_acc.at[idx_d.at[j]], sem, add=True)

        def c_wait(j, sem):
            pltpu.make_async_copy(ones_v, cnt_acc.at[idx_d.at[j]], sem).wait()

        c_start(0, c0)
        c_start(1, c1)

        @pl.loop(2, NCH, step=2)
        def _(j):
            c_wait(j - 2, c0)
            c_start(j, c0)
            c_wait(j - 1, c1)
            c_start(j + 1, c1)

        c_wait(NCH - 2, c0)
        c_wait(NCH - 1, c1)
        plsc.subcore_barrier()

        @pl.when(c == 0)
        def _():
            pltpu.sync_copy(cnt_acc.at[zsl], cnt_ui.at[zsl])

        @pl.when(c == 1)
        def _():
            pltpu.sync_copy(cnt_acc.at[zsl], cnt_iu.at[zsl])

    return cntk(eiu, eii, z16, o16)


def _agg(hu0, hu1, hi0, hi1, eiu, eii, z64):
    f32 = jnp.float32
    outs = [jax.ShapeDtypeStruct((NP, 64), f32)] * 4
    scratch = [
        pltpu.VMEM((NCH, CH), jnp.int32),
        pltpu.VMEM((NCH, CH), jnp.int32),
        pltpu.VMEM((CH, 32), jnp.int32),
        pltpu.VMEM((CH, 32), jnp.int32),
        pltpu.VMEM((CH, 64), f32),
        pltpu.VMEM((CH, 64), f32),
        pltpu.VMEM_SHARED((NP, 64), f32),
    ] + [pltpu.SemaphoreType.DMA] * 4

    @functools.partial(pl.kernel, out_type=outs, mesh=_get_mesh(),
                       scratch_types=scratch,
                       compiler_params=pltpu.CompilerParams(
                           use_tc_tiling_on_sc=False,
                           needs_layout_passes=False))
    def agg(hu0, hu1, hi0, hi1, eiu, eii, z64,
            sui0, sui1, siu0, siu1,
            idx_s, idx_d, bb0, bb1, fb0, fb1, acc,
            g0, g1, s0, s1):
        c = lax.axis_index("c")
        w = lax.axis_index("s")
        zsl = pl.ds(w * ZR, ZR)
        pltpu.sync_copy(z64, acc.at[zsl])
        plsc.subcore_barrier()

        @pl.when(c == 0)
        def _():
            _run_job(hu0, eiu, idx_s, idx_d, bb0, bb1, fb0, fb1, acc,
                     g0, g1, s0, s1)

        @pl.when(c == 1)
        def _():
            _run_job(hu1, eiu, idx_s, idx_d, bb0, bb1, fb0, fb1, acc,
                     g0, g1, s0, s1)

        plsc.subcore_barrier()

        @pl.when(c == 0)
        def _():
            pltpu.sync_copy(acc.at[zsl], sui0.at[zsl])

        @pl.when(c == 1)
        def _():
            pltpu.sync_copy(acc.at[zsl], sui1.at[zsl])

        pltpu.sync_copy(z64, acc.at[zsl])
        plsc.subcore_barrier()

        @pl.when(c == 0)
        def _():
            _run_job(hi0, eii, idx_s, idx_d, bb0, bb1, fb0, fb1, acc,
                     g0, g1, s0, s1)

        @pl.when(c == 1)
        def _():
            _run_job(hi1, eii, idx_s, idx_d, bb0, bb1, fb0, fb1, acc,
                     g0, g1, s0, s1)

        plsc.subcore_barrier()

        @pl.when(c == 0)
        def _():
            pltpu.sync_copy(acc.at[zsl], siu0.at[zsl])

        @pl.when(c == 1)
        def _():
            pltpu.sync_copy(acc.at[zsl], siu1.at[zsl])

    return agg(hu0, hu1, hi0, hi1, eiu, eii, z64)


_BS = 1000


def _pack64(x):
    b = jax.lax.bitcast_convert_type(x, jnp.int32) + jnp.int32(0x8000)
    lo = jax.lax.shift_right_logical(b[:, :32], 16)
    hi = b[:, 32:] & jnp.int32(-65536)
    return lo | hi


def _proj2(xu, Wu, bu, xi, Wi, bi):
    f32 = jnp.float32

    def body(xu_ref, wu_ref, bu_ref, xi_ref, wi_ref, bi_ref,
             u0_ref, u1_ref, i0_ref, i1_ref,
             u0p_ref, u1p_ref, i0p_ref, i1p_ref):
        hu = jnp.dot(xu_ref[...], wu_ref[...],
                     preferred_element_type=f32) + bu_ref[...]
        u0_ref[...] = hu[:, :64]
        u1_ref[...] = hu[:, 64:]
        u0p_ref[...] = _pack64(hu[:, :64])
        u1p_ref[...] = _pack64(hu[:, 64:])
        hi = jnp.dot(xi_ref[...], wi_ref[...],
                     preferred_element_type=f32) + bi_ref[...]
        i0_ref[...] = hi[:, :64]
        i1_ref[...] = hi[:, 64:]
        i0p_ref[...] = _pack64(hi[:, :64])
        i1p_ref[...] = _pack64(hi[:, 64:])

    row = pl.BlockSpec((_BS, D), lambda i: (i, 0))
    half = pl.BlockSpec((_BS, 64), lambda i: (i, 0))
    full = pl.BlockSpec((D, D), lambda i: (0, 0))
    bias = pl.BlockSpec((1, D), lambda i: (0, 0))
    packed = pl.BlockSpec((_BS, 32), lambda i: (i, 0))
    return pl.pallas_call(
        body,
        grid=(N // _BS,),
        in_specs=[row, full, bias, row, full, bias],
        out_specs=[half] * 4 + [packed] * 4,
        out_shape=[jax.ShapeDtypeStruct((N, 64), f32)] * 4
        + [jax.ShapeDtypeStruct((N, 32), jnp.int32)] * 4,
    )(xu, Wu, bu, xi, Wi, bi)


def _sage2_tc(a_in, b_in, relu, split):
    f32 = jnp.float32

    def one(refs, o_refs):
        s0_ref, s1_ref, c_ref, h0_ref, h1_ref, wl_ref, wr_ref, b_ref = refs
        cn = jnp.maximum(c_ref[:, 0:1], 1.0)
        m0 = s0_ref[...] / cn
        m1 = s1_ref[...] / cn
        o = (jnp.dot(m0, wl_ref[:64, :], preferred_element_type=f32)
             + jnp.dot(m1, wl_ref[64:, :], preferred_element_type=f32)
             + jnp.dot(h0_ref[...], wr_ref[:64, :], preferred_element_type=f32)
             + jnp.dot(h1_ref[...], wr_ref[64:, :], preferred_element_type=f32)
             + b_ref[...])
        if relu:
            o = jnp.maximum(o, 0.0)
        if split:
            o_refs[0][...] = o[:, :64]
            o_refs[1][...] = o[:, 64:]
            o_refs[2][...] = _pack64(o[:, :64])
            o_refs[3][...] = _pack64(o[:, 64:])
        else:
            o_refs[0][...] = o

    n_out = 4 if split else 1

    def body(*refs):
        ins, outs = refs[:16], refs[16:]
        one(ins[:8], outs[:n_out])
        one(ins[8:], outs[n_out:])

    half = pl.BlockSpec((_BS, 64), lambda i: (i, 0))
    cnt_spec = pl.BlockSpec((_BS, 16), lambda i: (i, 0))
    full = pl.BlockSpec((D, D), lambda i: (0, 0))
    bias = pl.BlockSpec((1, D), lambda i: (0, 0))
    one_in = [half, half, cnt_spec, half, half, full, full, bias]
    packed = pl.BlockSpec((_BS, 32), lambda i: (i, 0))
    if split:
        out_specs = ([half, half, packed, packed]
                     + [half, half, packed, packed])
        out_shape = ([jax.ShapeDtypeStruct((N, 64), f32)] * 2
                     + [jax.ShapeDtypeStruct((N, 32), jnp.int32)] * 2) * 2
    else:
        out_specs = [pl.BlockSpec((_BS, D), lambda i: (i, 0))] * 2
        out_shape = [jax.ShapeDtypeStruct((N, D), f32)] * 2

    return pl.pallas_call(
        body,
        grid=(N // _BS,),
        in_specs=one_in + one_in,
        out_specs=out_specs,
        out_shape=out_shape,
    )(*a_in, *b_in)


def _prep_edges(ei):
    ei = ei.astype(jnp.int32)
    pad = EP - E
    src = jnp.concatenate([ei[0], jnp.zeros((pad,), jnp.int32)])
    dummy = N + (jnp.arange(pad, dtype=jnp.int32) % (NP - N))
    dst = jnp.concatenate([ei[1], dummy])
    return jnp.stack([src, dst]).reshape(2, NS, NCH, CH)


def kernel(x_user, x_item,
           W_lin_user, b_lin_user, W_lin_item, b_lin_item,
           W1_l_ui, W1_r_ui, b1_ui, W1_l_iu, W1_r_iu, b1_iu,
           W2_l_ui, W2_r_ui, b2_ui, W2_l_iu, W2_r_iu, b2_iu,
           ei_ui, ei_iu):
    f32 = jnp.float32
    eiu = _prep_edges(ei_ui)
    eii = _prep_edges(ei_iu)
    z64 = jnp.zeros((ZR, 64), f32)
    z16 = jnp.zeros((ZR, 16), f32)
    o16 = jnp.ones((CH, 16), f32)

    cnt_ui, cnt_iu = _cnt_kernel(eiu, eii, z16, o16)
    (hu0, hu1, hi0, hi1, hu0p, hu1p, hi0p, hi1p) = _proj2(
        x_user, W_lin_user, b_lin_user.reshape(1, D),
        x_item, W_lin_item, b_lin_item.reshape(1, D))

    sui0, sui1, siu0, siu1 = _agg(hu0p, hu1p, hi0p, hi1p, eiu, eii, z64)

    (h2i0, h2i1, h2i0p, h2i1p, h2u0, h2u1, h2u0p, h2u1p) = _sage2_tc(
        (sui0, sui1, cnt_ui, hi0, hi1,
         W1_l_ui, W1_r_ui, b1_ui.reshape(1, D)),
        (siu0, siu1, cnt_iu, hu0, hu1,
         W1_l_iu, W1_r_iu, b1_iu.reshape(1, D)),
        relu=True, split=True)

    t_ui0, t_ui1, t_iu0, t_iu1 = _agg(h2u0p, h2u1p, h2i0p, h2i1p,
                                      eiu, eii, z64)

    out_i, out_u = _sage2_tc(
        (t_ui0, t_ui1, cnt_ui, h2i0, h2i1,
         W2_l_ui, W2_r_ui, b2_ui.reshape(1, D)),
        (t_iu0, t_iu1, cnt_iu, h2u0, h2u1,
         W2_l_iu, W2_r_iu, b2_iu.reshape(1, D)),
        relu=False, split=False)
    return out_u, out_i


# speedup vs baseline: 1.2249x; 1.2249x over previous
import functools

import jax
import jax.numpy as jnp
from jax import lax
from jax.experimental import pallas as pl
from jax.experimental.pallas import tpu as pltpu
from jax.experimental.pallas import tpu_sc as plsc

N = 10000
D = 128
E = 320000

NC = 2
NS = 16
CH = 128
NCH = 158
EP = NS * NCH * CH
NP = 10112
ZR = NP // NS

@functools.cache
def _get_mesh():
    return plsc.VectorSubcoreMesh(core_axis_name="c", subcore_axis_name="s",
                                  num_cores=NC)


def _run_job(table, ei, idx_s, idx_d, buf0, buf1, acc, g0, g1, s0, s1,
             cnt_kit=None):
    w = lax.axis_index("s")
    pltpu.sync_copy(ei.at[0, w], idx_s)
    pltpu.sync_copy(ei.at[1, w], idx_d)

    def g_start(j, buf, sem):
        pltpu.async_copy(table.at[idx_s.at[j]], buf, sem)

    def g_wait(j, buf, sem):
        pltpu.make_async_copy(table.at[idx_s.at[j]], buf, sem).wait()

    def s_start(j, buf, sem):
        pltpu.async_copy(buf, acc.at[idx_d.at[j]], sem, add=True)

    def s_wait(j, buf, sem):
        pltpu.make_async_copy(buf, acc.at[idx_d.at[j]], sem).wait()

    def c_start(j, sem):
        ones_v, cnt_acc = cnt_kit[0], cnt_kit[1]
        pltpu.async_copy(ones_v, cnt_acc.at[idx_d.at[j]], sem, add=True)

    def c_wait(j, sem):
        ones_v, cnt_acc = cnt_kit[0], cnt_kit[1]
        pltpu.make_async_copy(ones_v, cnt_acc.at[idx_d.at[j]], sem).wait()

    do_cnt = cnt_kit is not None
    if do_cnt:
        c0, c1 = cnt_kit[2], cnt_kit[3]

    g_start(0, buf0, g0)
    g_start(1, buf1, g1)

    @pl.loop(0, NCH - 2, step=2)
    def _(j):
        g_wait(j, buf0, g0)
        s_start(j, buf0, s0)
        if do_cnt:
            c_start(j, c0)
        g_wait(j + 1, buf1, g1)
        s_start(j + 1, buf1, s1)
        if do_cnt:
            c_start(j + 1, c1)
        s_wait(j, buf0, s0)
        if do_cnt:
            c_wait(j, c0)
        g_start(j + 2, buf0, g0)
        s_wait(j + 1, buf1, s1)
        if do_cnt:
            c_wait(j + 1, c1)
        g_start(j + 3, buf1, g1)

    j = NCH - 2
    g_wait(j, buf0, g0)
    s_start(j, buf0, s0)
    g_wait(j + 1, buf1, g1)
    s_start(j + 1, buf1, s1)
    s_wait(j, buf0, s0)
    s_wait(j + 1, buf1, s1)
    if do_cnt:
        c_start(j, c0)
        c_start(j + 1, c1)
        c_wait(j, c0)
        c_wait(j + 1, c1)


def _agg(hu0, hu1, hi0, hi1, eiu, eii, z64, z16=None, o16=None,
         with_cnt=False):
    f32 = jnp.float32
    outs = [jax.ShapeDtypeStruct((NP, 64), f32)] * 4
    scratch = [
        pltpu.VMEM((NCH, CH), jnp.int32),
        pltpu.VMEM((NCH, CH), jnp.int32),
        pltpu.VMEM((CH, 64), f32),
        pltpu.VMEM((CH, 64), f32),
        pltpu.VMEM_SHARED((NP, 64), f32),
    ] + [pltpu.SemaphoreType.DMA] * 4
    if with_cnt:
        outs = outs + [jax.ShapeDtypeStruct((NP, 16), f32)] * 2
        scratch = scratch + [
            pltpu.VMEM((CH, 16), f32),
            pltpu.VMEM_SHARED((NP, 16), f32),
            pltpu.SemaphoreType.DMA,
            pltpu.SemaphoreType.DMA,
        ]

    @functools.partial(pl.kernel, out_type=outs, mesh=_get_mesh(),
                       scratch_types=scratch,
                       compiler_params=pltpu.CompilerParams(
                           use_tc_tiling_on_sc=False))
    def agg(*refs):
        if with_cnt:
            (hu0, hu1, hi0, hi1, eiu, eii, z64, z16, o16,
             sui0, sui1, siu0, siu1, cnt_ui, cnt_iu,
             idx_s, idx_d, buf0, buf1, acc,
             g0, g1, s0, s1, ones_v, cnt_acc, c0, c1) = refs
        else:
            (hu0, hu1, hi0, hi1, eiu, eii, z64,
             sui0, sui1, siu0, siu1,
             idx_s, idx_d, buf0, buf1, acc,
             g0, g1, s0, s1) = refs
        c = lax.axis_index("c")
        w = lax.axis_index("s")
        zsl = pl.ds(w * ZR, ZR)
        pltpu.sync_copy(z64, acc.at[zsl])
        if with_cnt:
            pltpu.sync_copy(z16, cnt_acc.at[zsl])
            pltpu.sync_copy(o16, ones_v)
            kit = (ones_v, cnt_acc, c0, c1)
        plsc.subcore_barrier()

        @pl.when(c == 0)
        def _():
            _run_job(hu0, eiu, idx_s, idx_d, buf0, buf1, acc,
                     g0, g1, s0, s1,
                     cnt_kit=kit if with_cnt else None)

        @pl.when(c == 1)
        def _():
            _run_job(hu1, eiu, idx_s, idx_d, buf0, buf1, acc,
                     g0, g1, s0, s1)

        plsc.subcore_barrier()

        @pl.when(c == 0)
        def _():
            pltpu.sync_copy(acc.at[zsl], sui0.at[zsl])
            if with_cnt:
                pltpu.sync_copy(cnt_acc.at[zsl], cnt_ui.at[zsl])

        @pl.when(c == 1)
        def _():
            pltpu.sync_copy(acc.at[zsl], sui1.at[zsl])

        pltpu.sync_copy(z64, acc.at[zsl])
        if with_cnt:

            @pl.when(c == 1)
            def _():
                pltpu.sync_copy(z16, cnt_acc.at[zsl])

        plsc.subcore_barrier()

        @pl.when(c == 0)
        def _():
            _run_job(hi0, eii, idx_s, idx_d, buf0, buf1, acc,
                     g0, g1, s0, s1)

        @pl.when(c == 1)
        def _():
            _run_job(hi1, eii, idx_s, idx_d, buf0, buf1, acc,
                     g0, g1, s0, s1,
                     cnt_kit=kit if with_cnt else None)

        plsc.subcore_barrier()

        @pl.when(c == 0)
        def _():
            pltpu.sync_copy(acc.at[zsl], siu0.at[zsl])

        @pl.when(c == 1)
        def _():
            pltpu.sync_copy(acc.at[zsl], siu1.at[zsl])
            if with_cnt:
                pltpu.sync_copy(cnt_acc.at[zsl], cnt_iu.at[zsl])

    if with_cnt:
        return agg(hu0, hu1, hi0, hi1, eiu, eii, z64, z16, o16)
    return agg(hu0, hu1, hi0, hi1, eiu, eii, z64)


_BS = 1000


def _proj2(xu, Wu, bu, xi, Wi, bi):
    f32 = jnp.float32

    def body(xu_ref, wu_ref, bu_ref, xi_ref, wi_ref, bi_ref,
             u0_ref, u1_ref, i0_ref, i1_ref):
        hu = jnp.dot(xu_ref[...], wu_ref[...],
                     preferred_element_type=f32) + bu_ref[...]
        u0_ref[...] = hu[:, :64]
        u1_ref[...] = hu[:, 64:]
        hi = jnp.dot(xi_ref[...], wi_ref[...],
                     preferred_element_type=f32) + bi_ref[...]
        i0_ref[...] = hi[:, :64]
        i1_ref[...] = hi[:, 64:]

    row = pl.BlockSpec((_BS, D), lambda i: (i, 0))
    half = pl.BlockSpec((_BS, 64), lambda i: (i, 0))
    full = pl.BlockSpec((D, D), lambda i: (0, 0))
    bias = pl.BlockSpec((1, D), lambda i: (0, 0))
    return pl.pallas_call(
        body,
        grid=(N // _BS,),
        in_specs=[row, full, bias, row, full, bias],
        out_specs=[half, half, half, half],
        out_shape=[jax.ShapeDtypeStruct((N, 64), f32)] * 4,
    )(xu, Wu, bu, xi, Wi, bi)


def _sage2_tc(a_in, b_in, relu, split):
    f32 = jnp.float32

    def one(refs, o_refs):
        s0_ref, s1_ref, c_ref, h0_ref, h1_ref, wl_ref, wr_ref, b_ref = refs
        cn = jnp.maximum(c_ref[:, 0:1], 1.0)
        m0 = s0_ref[...] / cn
        m1 = s1_ref[...] / cn
        o = (jnp.dot(m0, wl_ref[:64, :], preferred_element_type=f32)
             + jnp.dot(m1, wl_ref[64:, :], preferred_element_type=f32)
             + jnp.dot(h0_ref[...], wr_ref[:64, :], preferred_element_type=f32)
             + jnp.dot(h1_ref[...], wr_ref[64:, :], preferred_element_type=f32)
             + b_ref[...])
        if relu:
            o = jnp.maximum(o, 0.0)
        if split:
            o_refs[0][...] = o[:, :64]
            o_refs[1][...] = o[:, 64:]
        else:
            o_refs[0][...] = o

    n_out = 2 if split else 1

    def body(*refs):
        ins, outs = refs[:16], refs[16:]
        one(ins[:8], outs[:n_out])
        one(ins[8:], outs[n_out:])

    half = pl.BlockSpec((_BS, 64), lambda i: (i, 0))
    cnt_spec = pl.BlockSpec((_BS, 16), lambda i: (i, 0))
    full = pl.BlockSpec((D, D), lambda i: (0, 0))
    bias = pl.BlockSpec((1, D), lambda i: (0, 0))
    one_in = [half, half, cnt_spec, half, half, full, full, bias]
    if split:
        out_specs = [half] * 4
        out_shape = [jax.ShapeDtypeStruct((N, 64), f32)] * 4
    else:
        out_specs = [pl.BlockSpec((_BS, D), lambda i: (i, 0))] * 2
        out_shape = [jax.ShapeDtypeStruct((N, D), f32)] * 2

    return pl.pallas_call(
        body,
        grid=(N // _BS,),
        in_specs=one_in + one_in,
        out_specs=out_specs,
        out_shape=out_shape,
    )(*a_in, *b_in)


def _prep_edges(ei):
    ei = ei.astype(jnp.int32)
    pad = EP - E
    src = jnp.concatenate([ei[0], jnp.zeros((pad,), jnp.int32)])
    dummy = N + (jnp.arange(pad, dtype=jnp.int32) % (NP - N))
    dst = jnp.concatenate([ei[1], dummy])
    return jnp.stack([src, dst]).reshape(2, NS, NCH, CH)


def kernel(x_user, x_item,
           W_lin_user, b_lin_user, W_lin_item, b_lin_item,
           W1_l_ui, W1_r_ui, b1_ui, W1_l_iu, W1_r_iu, b1_iu,
           W2_l_ui, W2_r_ui, b2_ui, W2_l_iu, W2_r_iu, b2_iu,
           ei_ui, ei_iu):
    f32 = jnp.float32
    eiu = _prep_edges(ei_ui)
    eii = _prep_edges(ei_iu)
    z64 = jnp.zeros((ZR, 64), f32)
    z16 = jnp.zeros((ZR, 16), f32)
    o16 = jnp.ones((CH, 16), f32)

    hu0, hu1, hi0, hi1 = _proj2(x_user, W_lin_user, b_lin_user.reshape(1, D),
                                x_item, W_lin_item, b_lin_item.reshape(1, D))

    sui0, sui1, siu0, siu1, cnt_ui, cnt_iu = _agg(
        hu0, hu1, hi0, hi1, eiu, eii, z64, z16, o16, with_cnt=True)

    h2i0, h2i1, h2u0, h2u1 = _sage2_tc(
        (sui0, sui1, cnt_ui, hi0, hi1,
         W1_l_ui, W1_r_ui, b1_ui.reshape(1, D)),
        (siu0, siu1, cnt_iu, hu0, hu1,
         W1_l_iu, W1_r_iu, b1_iu.reshape(1, D)),
        relu=True, split=True)

    t_ui0, t_ui1, t_iu0, t_iu1 = _agg(h2u0, h2u1, h2i0, h2i1,
                                      eiu, eii, z64)

    out_i, out_u = _sage2_tc(
        (t_ui0, t_ui1, cnt_ui, h2i0, h2i1,
         W2_l_ui, W2_r_ui, b2_ui.reshape(1, D)),
        (t_iu0, t_iu1, cnt_iu, h2u0, h2u1,
         W2_l_iu, W2_r_iu, b2_iu.reshape(1, D)),
        relu=False, split=False)
    return out_u, out_i
